# baseline (speedup 1.0000x reference)
import functools

import jax
import jax.numpy as jnp
from jax import lax
from jax.experimental import pallas as pl
from jax.experimental.pallas import tpu as pltpu, tpu_sc as plsc

_D = 32
_M = 256
_C = 512
_HALF = 4096


def _argmin_body(n_embeddings, x_ref, e_ref, x2_ref, e2_ref, o_ref):
    x2 = x2_ref[...]
    xb = x_ref[...].astype(jnp.bfloat16)

    def half_argmin(base):
        best_d = jnp.full((_M, 1), jnp.inf, jnp.float32)
        best_i = jnp.zeros((_M, 1), jnp.int32)
        for c in range(_HALF // _C):
            off = base + c * _C
            e = e_ref[:, pl.ds(off, _C)]
            e2 = e2_ref[:, pl.ds(off, _C)]
            mm = jnp.dot(xb, e, preferred_element_type=jnp.float32)
            d = x2 + e2 - 2.0 * mm
            lv = jnp.min(d, axis=1, keepdims=True)
            li = jnp.argmin(d, axis=1).astype(jnp.int32).reshape(_M, 1) + off
            upd = lv < best_d
            best_d = jnp.where(upd, lv, best_d)
            best_i = jnp.where(upd, li, best_i)
        return best_d, best_i

    vl, il = half_argmin(0)
    vr, ir = half_argmin(_HALF)
    pick_r = vr < vl.astype(jnp.bfloat16).astype(jnp.float32)
    o_ref[...] = jnp.where(pick_r, ir, il)


def _make_gather(n_tokens, row):
    info = plsc.get_sparse_core_info()
    nw = info.num_cores * info.num_subcores
    b_per_w = n_tokens // nw
    chunk = 256
    mesh = plsc.VectorSubcoreMesh(core_axis_name="c", subcore_axis_name="s")

    @functools.partial(
        pl.kernel, mesh=mesh,
        out_type=jax.ShapeDtypeStruct((n_tokens, row), jnp.float32),
        scratch_types=[
            pltpu.VMEM((b_per_w,), jnp.int32),
            pltpu.VMEM((chunk, row), jnp.float32),
            pltpu.SemaphoreType.DMA,
        ],
    )
    def gather(table_hbm, idx_hbm, out_hbm, idx_v, rows_v, sem):
        wid = lax.axis_index("s") * info.num_cores + lax.axis_index("c")
        base = wid * b_per_w
        pltpu.sync_copy(idx_hbm.at[pl.ds(base, b_per_w)], idx_v)
        for c in range(b_per_w // chunk):
            pltpu.async_copy(
                table_hbm.at[idx_v.at[pl.ds(c * chunk, chunk)]],
                rows_v, sem).wait()
            pltpu.sync_copy(rows_v,
                            out_hbm.at[pl.ds(base + c * chunk, chunk)])

    return gather


def kernel(inputs, embeddings):
    d = inputs.shape[-1]
    n_embeddings = embeddings.shape[1]
    flat = inputs.reshape(-1, d)
    n_tokens = flat.shape[0]
    x2 = jnp.sum(flat ** 2, axis=1, keepdims=True)
    e2 = jnp.sum(embeddings ** 2, axis=0, keepdims=True)
    ebf = embeddings.astype(jnp.bfloat16)
    idx = pl.pallas_call(
        functools.partial(_argmin_body, n_embeddings),
        grid=(n_tokens // _M,),
        in_specs=[
            pl.BlockSpec((_M, d), lambda i: (i, 0)),
            pl.BlockSpec((d, n_embeddings), lambda i: (0, 0)),
            pl.BlockSpec((_M, 1), lambda i: (i, 0)),
            pl.BlockSpec((1, n_embeddings), lambda i: (0, 0)),
        ],
        out_specs=pl.BlockSpec((_M, 1), lambda i: (i, 0)),
        out_shape=jax.ShapeDtypeStruct((n_tokens, 1), jnp.int32),
    )(flat, ebf, x2, e2)
    table = jnp.pad(embeddings.T, ((0, 0), (0, 128 - d)))
    q = _make_gather(n_tokens, 128)(table, idx.reshape(n_tokens))[:, :d]
    out = flat + (q - flat)
    return out.reshape(inputs.shape)


# speedup vs baseline: 1.8062x; 1.8062x over previous
import functools

import jax
import jax.numpy as jnp
from jax import lax
from jax.experimental import pallas as pl
from jax.experimental.pallas import tpu as pltpu, tpu_sc as plsc

_D = 32
_M = 256
_C = 1024
_HALF = 4096


def _argmin_body(n_embeddings, x_ref, e_ref, x2_ref, e2_ref, o_ref):
    x2 = x2_ref[...]
    xb = x_ref[...].astype(jnp.bfloat16)

    def half_argmin(base):
        best_d = jnp.full((_M, 1), jnp.inf, jnp.float32)
        best_i = jnp.zeros((_M, 1), jnp.int32)
        for c in range(_HALF // _C):
            off = base + c * _C
            e = e_ref[:, pl.ds(off, _C)]
            e2 = e2_ref[:, pl.ds(off, _C)]
            mm = jnp.dot(xb, e, preferred_element_type=jnp.float32)
            d = x2 + e2 - 2.0 * mm
            lv = jnp.min(d, axis=1, keepdims=True)
            li = jnp.argmin(d, axis=1).astype(jnp.int32).reshape(_M, 1) + off
            upd = lv < best_d
            best_d = jnp.where(upd, lv, best_d)
            best_i = jnp.where(upd, li, best_i)
        return best_d, best_i

    vl, il = half_argmin(0)
    vr, ir = half_argmin(_HALF)
    pick_r = vr < vl.astype(jnp.bfloat16).astype(jnp.float32)
    o_ref[...] = jnp.where(pick_r, ir, il)


def _make_gather(n_tokens, row):
    info = plsc.get_sparse_core_info()
    nw = info.num_cores * info.num_subcores
    b_per_w = n_tokens // nw
    chunk = 256
    mesh = plsc.VectorSubcoreMesh(core_axis_name="c", subcore_axis_name="s")

    @functools.partial(
        pl.kernel, mesh=mesh,
        out_type=jax.ShapeDtypeStruct((n_tokens, row), jnp.float32),
        scratch_types=[
            pltpu.VMEM((b_per_w,), jnp.int32),
            pltpu.VMEM((chunk, row), jnp.float32),
            pltpu.SemaphoreType.DMA,
        ],
    )
    def gather(table_hbm, idx_hbm, out_hbm, idx_v, rows_v, sem):
        wid = lax.axis_index("s") * info.num_cores + lax.axis_index("c")
        base = wid * b_per_w
        pltpu.sync_copy(idx_hbm.at[pl.ds(base, b_per_w)], idx_v)
        for c in range(b_per_w // chunk):
            pltpu.async_copy(
                table_hbm.at[idx_v.at[pl.ds(c * chunk, chunk)]],
                rows_v, sem).wait()
            pltpu.sync_copy(rows_v,
                            out_hbm.at[pl.ds(base + c * chunk, chunk)])

    return gather


def kernel(inputs, embeddings):
    d = inputs.shape[-1]
    n_embeddings = embeddings.shape[1]
    flat = inputs.reshape(-1, d)
    n_tokens = flat.shape[0]
    x2 = jnp.sum(flat ** 2, axis=1, keepdims=True)
    e2 = jnp.sum(embeddings ** 2, axis=0, keepdims=True)
    ebf = embeddings.astype(jnp.bfloat16)
    idx = pl.pallas_call(
        functools.partial(_argmin_body, n_embeddings),
        grid=(n_tokens // _M,),
        in_specs=[
            pl.BlockSpec((_M, d), lambda i: (i, 0)),
            pl.BlockSpec((d, n_embeddings), lambda i: (0, 0)),
            pl.BlockSpec((_M, 1), lambda i: (i, 0)),
            pl.BlockSpec((1, n_embeddings), lambda i: (0, 0)),
        ],
        out_specs=pl.BlockSpec((_M, 1), lambda i: (i, 0)),
        out_shape=jax.ShapeDtypeStruct((n_tokens, 1), jnp.int32),
    )(flat, ebf, x2, e2)
    table = jnp.pad(embeddings.T, ((0, 0), (0, 128 - d)))
    q = _make_gather(n_tokens, 128)(table, idx.reshape(n_tokens))[:, :d]
    out = flat + (q - flat)
    return out.reshape(inputs.shape)


# speedup vs baseline: 4.1301x; 2.2866x over previous
import functools

import jax
import jax.numpy as jnp
from jax import lax
from jax.experimental import pallas as pl
from jax.experimental.pallas import tpu as pltpu, tpu_sc as plsc

_D = 32
_M = 256
_C = 2048
_HALF = 4096


def _argmin_body(n_embeddings, x_ref, e_ref, x2_ref, e2_ref, o_ref):
    x2 = x2_ref[...]
    xb = x_ref[...].astype(jnp.bfloat16)

    def half_argmin(base):
        best_d = jnp.full((_M, 1), jnp.inf, jnp.float32)
        best_i = jnp.zeros((_M, 1), jnp.int32)
        for c in range(_HALF // _C):
            off = base + c * _C
            e = e_ref[:, pl.ds(off, _C)]
            e2 = e2_ref[:, pl.ds(off, _C)]
            mm = jnp.dot(xb, e, preferred_element_type=jnp.float32)
            d = x2 + e2 - 2.0 * mm
            lv = jnp.min(d, axis=1, keepdims=True)
            li = jnp.argmin(d, axis=1).astype(jnp.int32).reshape(_M, 1) + off
            upd = lv < best_d
            best_d = jnp.where(upd, lv, best_d)
            best_i = jnp.where(upd, li, best_i)
        return best_d, best_i

    vl, il = half_argmin(0)
    vr, ir = half_argmin(_HALF)
    pick_r = vr < vl.astype(jnp.bfloat16).astype(jnp.float32)
    o_ref[...] = jnp.where(pick_r, ir, il)


def _make_gather(n_tokens, row):
    info = plsc.get_sparse_core_info()
    nw = info.num_cores * info.num_subcores
    b_per_w = n_tokens // nw
    chunk = 256
    mesh = plsc.VectorSubcoreMesh(core_axis_name="c", subcore_axis_name="s")

    @functools.partial(
        pl.kernel, mesh=mesh,
        out_type=jax.ShapeDtypeStruct((n_tokens, row), jnp.float32),
        scratch_types=[
            pltpu.VMEM((b_per_w,), jnp.int32),
            pltpu.VMEM((chunk, row), jnp.float32),
            pltpu.SemaphoreType.DMA,
        ],
    )
    def gather(table_hbm, idx_hbm, out_hbm, idx_v, rows_v, sem):
        wid = lax.axis_index("s") * info.num_cores + lax.axis_index("c")
        base = wid * b_per_w
        pltpu.sync_copy(idx_hbm.at[pl.ds(base, b_per_w)], idx_v)
        for c in range(b_per_w // chunk):
            pltpu.async_copy(
                table_hbm.at[idx_v.at[pl.ds(c * chunk, chunk)]],
                rows_v, sem).wait()
            pltpu.sync_copy(rows_v,
                            out_hbm.at[pl.ds(base + c * chunk, chunk)])

    return gather


def kernel(inputs, embeddings):
    d = inputs.shape[-1]
    n_embeddings = embeddings.shape[1]
    flat = inputs.reshape(-1, d)
    n_tokens = flat.shape[0]
    x2 = jnp.sum(flat ** 2, axis=1, keepdims=True)
    e2 = jnp.sum(embeddings ** 2, axis=0, keepdims=True)
    ebf = embeddings.astype(jnp.bfloat16)
    idx = pl.pallas_call(
        functools.partial(_argmin_body, n_embeddings),
        grid=(n_tokens // _M,),
        in_specs=[
            pl.BlockSpec((_M, d), lambda i: (i, 0)),
            pl.BlockSpec((d, n_embeddings), lambda i: (0, 0)),
            pl.BlockSpec((_M, 1), lambda i: (i, 0)),
            pl.BlockSpec((1, n_embeddings), lambda i: (0, 0)),
        ],
        out_specs=pl.BlockSpec((_M, 1), lambda i: (i, 0)),
        out_shape=jax.ShapeDtypeStruct((n_tokens, 1), jnp.int32),
    )(flat, ebf, x2, e2)
    table = jnp.pad(embeddings.T, ((0, 0), (0, 128 - d)))
    q = _make_gather(n_tokens, 128)(table, idx.reshape(n_tokens))[:, :d]
    out = flat + (q - flat)
    return out.reshape(inputs.shape)


# speedup vs baseline: 4.1416x; 1.0028x over previous
import functools

import jax
import jax.numpy as jnp
from jax import lax
from jax.experimental import pallas as pl
from jax.experimental.pallas import tpu as pltpu, tpu_sc as plsc

_D = 32
_M = 256
_C = 4096
_HALF = 4096


def _argmin_body(n_embeddings, x_ref, e_ref, x2_ref, e2_ref, o_ref):
    x2 = x2_ref[...]
    xb = x_ref[...].astype(jnp.bfloat16)

    def half_argmin(base):
        best_d = jnp.full((_M, 1), jnp.inf, jnp.float32)
        best_i = jnp.zeros((_M, 1), jnp.int32)
        for c in range(_HALF // _C):
            off = base + c * _C
            e = e_ref[:, pl.ds(off, _C)]
            e2 = e2_ref[:, pl.ds(off, _C)]
            mm = jnp.dot(xb, e, preferred_element_type=jnp.float32)
            d = x2 + e2 - 2.0 * mm
            lv = jnp.min(d, axis=1, keepdims=True)
            li = jnp.argmin(d, axis=1).astype(jnp.int32).reshape(_M, 1) + off
            upd = lv < best_d
            best_d = jnp.where(upd, lv, best_d)
            best_i = jnp.where(upd, li, best_i)
        return best_d, best_i

    vl, il = half_argmin(0)
    vr, ir = half_argmin(_HALF)
    pick_r = vr < vl.astype(jnp.bfloat16).astype(jnp.float32)
    o_ref[...] = jnp.where(pick_r, ir, il)


def _make_gather(n_tokens, row):
    info = plsc.get_sparse_core_info()
    nw = info.num_cores * info.num_subcores
    b_per_w = n_tokens // nw
    chunk = 256
    mesh = plsc.VectorSubcoreMesh(core_axis_name="c", subcore_axis_name="s")

    @functools.partial(
        pl.kernel, mesh=mesh,
        out_type=jax.ShapeDtypeStruct((n_tokens, row), jnp.float32),
        scratch_types=[
            pltpu.VMEM((b_per_w,), jnp.int32),
            pltpu.VMEM((chunk, row), jnp.float32),
            pltpu.SemaphoreType.DMA,
        ],
    )
    def gather(table_hbm, idx_hbm, out_hbm, idx_v, rows_v, sem):
        wid = lax.axis_index("s") * info.num_cores + lax.axis_index("c")
        base = wid * b_per_w
        pltpu.sync_copy(idx_hbm.at[pl.ds(base, b_per_w)], idx_v)
        for c in range(b_per_w // chunk):
            pltpu.async_copy(
                table_hbm.at[idx_v.at[pl.ds(c * chunk, chunk)]],
                rows_v, sem).wait()
            pltpu.sync_copy(rows_v,
                            out_hbm.at[pl.ds(base + c * chunk, chunk)])

    return gather


def kernel(inputs, embeddings):
    d = inputs.shape[-1]
    n_embeddings = embeddings.shape[1]
    flat = inputs.reshape(-1, d)
    n_tokens = flat.shape[0]
    x2 = jnp.sum(flat ** 2, axis=1, keepdims=True)
    e2 = jnp.sum(embeddings ** 2, axis=0, keepdims=True)
    ebf = embeddings.astype(jnp.bfloat16)
    idx = pl.pallas_call(
        functools.partial(_argmin_body, n_embeddings),
        grid=(n_tokens // _M,),
        in_specs=[
            pl.BlockSpec((_M, d), lambda i: (i, 0)),
            pl.BlockSpec((d, n_embeddings), lambda i: (0, 0)),
            pl.BlockSpec((_M, 1), lambda i: (i, 0)),
            pl.BlockSpec((1, n_embeddings), lambda i: (0, 0)),
        ],
        out_specs=pl.BlockSpec((_M, 1), lambda i: (i, 0)),
        out_shape=jax.ShapeDtypeStruct((n_tokens, 1), jnp.int32),
    )(flat, ebf, x2, e2)
    table = jnp.pad(embeddings.T, ((0, 0), (0, 128 - d)))
    q = _make_gather(n_tokens, 128)(table, idx.reshape(n_tokens))[:, :d]
    out = flat + (q - flat)
    return out.reshape(inputs.shape)
